# baseline (speedup 1.0000x reference)
import functools

import jax
import jax.numpy as jnp
from jax import lax
from jax.experimental import pallas as pl
from jax.experimental.pallas import tpu as pltpu
from jax.experimental.pallas import tpu_sc as plsc

BSZ, BEAM, V = 32, 8, 32768
K = 2 * BEAM
NROW = BSZ * BEAM
NW = 32
RPW = NROW // NW
NGRP = V // 256
NEG = float("-inf")
BIG = 1 << 30


def _sc_body(lp0_hbm, lp1_hbm, scb0_hbm, scb1_hbm,
             tv0_hbm, ti0_hbm, tv1_hbm, ti1_hbm,
             rowA, rowB, gms, tidx_v, scv0, scv1,
             otv0, oti0, otv1, oti1, semA, semB):
    wid = lax.axis_index("s") * 2 + lax.axis_index("c")
    base = wid * RPW
    lane = lax.iota(jnp.int32, K)
    zeros16 = jnp.zeros((K,), jnp.int32)

    pltpu.sync_copy(scb0_hbm.at[pl.ds(base * K, RPW * K)], scv0)
    pltpu.sync_copy(scb1_hbm.at[pl.ds(base * K, RPW * K)], scv1)

    def scan_row(row_v, s, obase, otv, oti):
        def pass_a(g, carry):
            off = g * 256
            ms = [row_v[pl.ds(off + c * K, K)] for c in range(16)]
            while len(ms) > 1:
                ms = [jnp.maximum(ms[i], ms[i + 1])
                      for i in range(0, len(ms), 2)]
            gm = jnp.max(ms[0])
            gms[pl.ds(g * K, K)] = jnp.full((K,), gm, jnp.float32)
            return carry

        lax.fori_loop(0, NGRP, pass_a, 0)

        Ts = Ti = None
        for k in range(8):
            gmk = plsc.load_gather(gms, [lane * 17 + 256 * k])
            gik = lane + 16 * k
            sv, si = plsc.sort_key_val(gmk, gik, descending=True)
            if Ts is None:
                Ts, Ti = sv, si
            else:
                bv = lax.rev(sv, (0,))
                bi = lax.rev(si, (0,))
                keep = (Ts > bv) | ((Ts == bv) & (Ti < bi))
                Ts, Ti = plsc.sort_key_val(jnp.where(keep, Ts, bv),
                                           jnp.where(keep, Ti, bi),
                                           descending=True)
        tidx_v[...] = Ti

        def hot(r, carry):
            cv, ci = carry
            gsp = plsc.load_gather(tidx_v, [zeros16 + r])
            g = gsp[0]
            off = g * 256
            leaves = []
            for c in range(16):
                x = row_v[pl.ds(off + c * K, K)] + s
                xi = lane + (off + c * K)
                leaves.append(plsc.sort_key_val(x, xi, descending=True))

            def mrg(a, b):
                av, ai = a
                bv = lax.rev(b[0], (0,))
                bi = lax.rev(b[1], (0,))
                keep = (av > bv) | ((av == bv) & (ai < bi))
                return plsc.sort_key_val(jnp.where(keep, av, bv),
                                         jnp.where(keep, ai, bi),
                                         descending=True)

            while len(leaves) > 1:
                leaves = [mrg(leaves[i], leaves[i + 1])
                          for i in range(0, len(leaves), 2)]
            gv, gi = leaves[0]
            keep = (cv > gv) | ((cv == gv) & (ci < gi))
            mv = jnp.where(keep, cv, gv)
            mi = jnp.where(keep, ci, gi)
            cv, ci = plsc.sort_key_val(mv, mi, descending=False)
            return cv, ci

        cv0 = jnp.full((K,), NEG, jnp.float32)
        ci0 = jnp.zeros((K,), jnp.int32)
        cv, ci = lax.fori_loop(0, 16, hot, (cv0, ci0))
        otv[pl.ds(obase, K)] = lax.rev(cv, (0,))
        oti[pl.ds(obase, K)] = lax.rev(ci, (0,))

    pltpu.async_copy(lp0_hbm.at[base], rowA, semA)

    def rowloop(j, carry):
        row = base + j
        pltpu.make_async_copy(lp0_hbm.at[0], rowA, semA).wait()
        pltpu.async_copy(lp1_hbm.at[row], rowB, semB)
        scan_row(rowA, scv0[pl.ds(j * K, K)], j * K, otv0, oti0)
        pltpu.make_async_copy(lp1_hbm.at[0], rowB, semB).wait()
        nxt = jnp.minimum(row + 1, NROW - 1)
        pltpu.async_copy(lp0_hbm.at[nxt], rowA, semA)
        scan_row(rowB, scv1[pl.ds(j * K, K)], j * K, otv1, oti1)
        return carry

    lax.fori_loop(0, RPW, rowloop, 0)
    pltpu.make_async_copy(lp0_hbm.at[0], rowA, semA).wait()

    pltpu.sync_copy(otv0, tv0_hbm.at[pl.ds(base * K, RPW * K)])
    pltpu.sync_copy(oti0, ti0_hbm.at[pl.ds(base * K, RPW * K)])
    pltpu.sync_copy(otv1, tv1_hbm.at[pl.ds(base * K, RPW * K)])
    pltpu.sync_copy(oti1, ti1_hbm.at[pl.ds(base * K, RPW * K)])


def _sc_topk(lp0, lp1, scb0, scb1):
    f32 = jnp.float32
    i32 = jnp.int32
    run = pl.kernel(
        _sc_body,
        out_type=(
            jax.ShapeDtypeStruct((NROW * K,), f32),
            jax.ShapeDtypeStruct((NROW * K,), i32),
            jax.ShapeDtypeStruct((NROW * K,), f32),
            jax.ShapeDtypeStruct((NROW * K,), i32),
        ),
        mesh=plsc.VectorSubcoreMesh(core_axis_name="c", subcore_axis_name="s"),
        compiler_params=pltpu.CompilerParams(needs_layout_passes=False),
        scratch_types=[
            pltpu.VMEM((V,), f32),
            pltpu.VMEM((V,), f32),
            pltpu.VMEM((V // 16,), f32),
            pltpu.VMEM((K,), i32),
            pltpu.VMEM((RPW * K,), f32),
            pltpu.VMEM((RPW * K,), f32),
            pltpu.VMEM((RPW * K,), f32),
            pltpu.VMEM((RPW * K,), i32),
            pltpu.VMEM((RPW * K,), f32),
            pltpu.VMEM((RPW * K,), i32),
            pltpu.SemaphoreType.DMA,
            pltpu.SemaphoreType.DMA,
        ],
    )
    return run(lp0, lp1, scb0, scb1)


def _combine_body(tv0_ref, ti0_ref, tv1_ref, ti1_ref,
                  s0_ref, s1_ref, t0_ref, t1_ref, ib_ref):
    lane4 = jax.lax.broadcasted_iota(jnp.int32, (1, 1, 1, K), 3)

    def rerank(va, ia):
        ve = va[:, :, :, None]
        vf = va[:, :, None, :]
        ie = ia[:, :, :, None]
        if_ = ia[:, :, None, :]
        cond = (vf > ve) | ((vf == ve) & (if_ < ie))
        rank = jnp.sum(cond.astype(jnp.int32), axis=3)
        oh = rank[:, :, :, None] == lane4
        vs = jnp.sum(jnp.where(oh, ve, 0.0), axis=2)
        is_ = jnp.sum(jnp.where(oh, ie, 0), axis=2)
        return vs, is_

    tv0, ti0 = rerank(tv0_ref[...], ti0_ref[...])
    tv1, ti1 = rerank(tv1_ref[...], ti1_ref[...])

    ss = (tv0[:, :, :, None] + tv1[:, :, None, :]).reshape(BSZ, BEAM * K * K)
    fidx = jax.lax.broadcasted_iota(jnp.int32, (BSZ, BEAM * K * K), 1)
    lane16 = jax.lax.broadcasted_iota(jnp.int32, (1, K), 1)

    vacc = jnp.zeros((BSZ, K), jnp.float32)
    iacc = jnp.zeros((BSZ, K), jnp.int32)
    for t in range(K):
        m = jnp.max(ss, axis=1, keepdims=True)
        idx = jnp.min(jnp.where(ss == m, fidx, BIG), axis=1,
                      keepdims=True)
        ss = jnp.where(fidx == idx, NEG, ss)
        oh = lane16 == t
        vacc = vacc + jnp.where(oh, m, 0.0)
        iacc = iacc + jnp.where(oh, idx, 0)

    ib = iacc >> 8
    rem = iacc & 255
    i0 = rem >> 4
    i1 = rem & 15

    beam_i = jax.lax.broadcasted_iota(jnp.int32, (1, 1, BEAM, K), 2)
    col_i = jax.lax.broadcasted_iota(jnp.int32, (1, 1, BEAM, K), 3)
    sel0 = ((ib[:, :, None, None] == beam_i)
            & (i0[:, :, None, None] == col_i))
    sel1 = ((ib[:, :, None, None] == beam_i)
            & (i1[:, :, None, None] == col_i))
    s0_ref[...] = jnp.sum(jnp.where(sel0, tv0[:, None], 0.0), axis=(2, 3))
    s1_ref[...] = jnp.sum(jnp.where(sel1, tv1[:, None], 0.0), axis=(2, 3))
    t0_ref[...] = jnp.sum(jnp.where(sel0, ti0[:, None], 0), axis=(2, 3))
    t1_ref[...] = jnp.sum(jnp.where(sel1, ti1[:, None], 0), axis=(2, 3))
    ib_ref[...] = ib


def _tc_combine(tv0, ti0, tv1, ti1):
    out_shapes = tuple(
        jax.ShapeDtypeStruct((BSZ, K), dt)
        for dt in (jnp.float32, jnp.float32, jnp.int32, jnp.int32, jnp.int32))
    return pl.pallas_call(
        _combine_body,
        out_shape=out_shapes,
    )(tv0, ti0, tv1, ti1)


def kernel(step, lprobs_ch0, lprobs_ch1, scores_ch0, scores_ch1):
    sc0 = jax.lax.dynamic_index_in_dim(scores_ch0, step - 1, axis=2,
                                       keepdims=False)
    sc1 = jax.lax.dynamic_index_in_dim(scores_ch1, step - 1, axis=2,
                                       keepdims=False)
    lp0 = lprobs_ch0.reshape(NROW, V)
    lp1 = lprobs_ch1.reshape(NROW, V)
    scb0 = jnp.broadcast_to(sc0.reshape(NROW, 1), (NROW, K)).reshape(NROW * K)
    scb1 = jnp.broadcast_to(sc1.reshape(NROW, 1), (NROW, K)).reshape(NROW * K)

    tv0, ti0, tv1, ti1 = _sc_topk(lp0, lp1, scb0, scb1)
    tv0 = tv0.reshape(NROW, K)
    ti0 = ti0.reshape(NROW, K)
    tv1 = tv1.reshape(NROW, K)
    ti1 = ti1.reshape(NROW, K)

    s0, s1, t0, t1, ib = _tc_combine(
        tv0.reshape(BSZ, BEAM, K), ti0.reshape(BSZ, BEAM, K),
        tv1.reshape(BSZ, BEAM, K), ti1.reshape(BSZ, BEAM, K))
    return (s0, s1, t0, t1, ib)


# speedup vs baseline: 1.2214x; 1.2214x over previous
import functools

import jax
import jax.numpy as jnp
from jax import lax
from jax.experimental import pallas as pl
from jax.experimental.pallas import tpu as pltpu
from jax.experimental.pallas import tpu_sc as plsc

BSZ, BEAM, V = 32, 8, 32768
K = 2 * BEAM
NROW = BSZ * BEAM
NW = 32
RPW = NROW // NW
NGRP = V // 256
NEG = float("-inf")
BIG = 1 << 30


def _sc_body(lp0_hbm, lp1_hbm, scb0_hbm, scb1_hbm,
             tv0_hbm, ti0_hbm, tv1_hbm, ti1_hbm,
             rowA, rowB, gms, tidx_v, scv0, scv1,
             otv0, oti0, otv1, oti1, semA, semB):
    wid = lax.axis_index("s") * 2 + lax.axis_index("c")
    base = wid * RPW
    lane = lax.iota(jnp.int32, K)
    zeros16 = jnp.zeros((K,), jnp.int32)

    pltpu.sync_copy(scb0_hbm.at[pl.ds(base * K, RPW * K)], scv0)
    pltpu.sync_copy(scb1_hbm.at[pl.ds(base * K, RPW * K)], scv1)

    def scan_row(row_v, s, obase, otv, oti):
        @plsc.parallel_loop(0, NGRP, 1, unroll=2)
        def pass_a(g):
            off = g * 256
            ms = [row_v[pl.ds(off + c * K, K)] for c in range(16)]
            while len(ms) > 1:
                ms = [jnp.maximum(ms[i], ms[i + 1])
                      for i in range(0, len(ms), 2)]
            gm = jnp.max(ms[0])
            gms[pl.ds(g * K, K)] = jnp.full((K,), gm, jnp.float32)

        Ts = Ti = None
        for k in range(8):
            gmk = plsc.load_gather(gms, [lane * 17 + 256 * k])
            gik = lane + 16 * k
            sv, si = plsc.sort_key_val(gmk, gik, descending=True)
            if Ts is None:
                Ts, Ti = sv, si
            else:
                bv = lax.rev(sv, (0,))
                bi = lax.rev(si, (0,))
                keep = (Ts > bv) | ((Ts == bv) & (Ti < bi))
                Ts, Ti = plsc.sort_key_val(jnp.where(keep, Ts, bv),
                                           jnp.where(keep, Ti, bi),
                                           descending=True)
        tidx_v[...] = Ti

        def hot(r, carry):
            cv, ci = carry
            gsp = plsc.load_gather(tidx_v, [zeros16 + r])
            g = gsp[0]
            off = g * 256
            leaves = []
            for c in range(16):
                x = row_v[pl.ds(off + c * K, K)] + s
                xi = lane + (off + c * K)
                leaves.append(plsc.sort_key_val(x, xi, descending=True))

            def mrg(a, b):
                av, ai = a
                bv = lax.rev(b[0], (0,))
                bi = lax.rev(b[1], (0,))
                keep = (av > bv) | ((av == bv) & (ai < bi))
                return plsc.sort_key_val(jnp.where(keep, av, bv),
                                         jnp.where(keep, ai, bi),
                                         descending=True)

            while len(leaves) > 1:
                leaves = [mrg(leaves[i], leaves[i + 1])
                          for i in range(0, len(leaves), 2)]
            gv, gi = leaves[0]
            keep = (cv > gv) | ((cv == gv) & (ci < gi))
            mv = jnp.where(keep, cv, gv)
            mi = jnp.where(keep, ci, gi)
            cv, ci = plsc.sort_key_val(mv, mi, descending=False)
            return cv, ci

        cv0 = jnp.full((K,), NEG, jnp.float32)
        ci0 = jnp.zeros((K,), jnp.int32)
        cv, ci = lax.fori_loop(0, 16, hot, (cv0, ci0))
        otv[pl.ds(obase, K)] = lax.rev(cv, (0,))
        oti[pl.ds(obase, K)] = lax.rev(ci, (0,))

    pltpu.async_copy(lp0_hbm.at[base], rowA, semA)

    def rowloop(j, carry):
        row = base + j
        pltpu.make_async_copy(lp0_hbm.at[0], rowA, semA).wait()
        pltpu.async_copy(lp1_hbm.at[row], rowB, semB)
        scan_row(rowA, scv0[pl.ds(j * K, K)], j * K, otv0, oti0)
        pltpu.make_async_copy(lp1_hbm.at[0], rowB, semB).wait()
        nxt = jnp.minimum(row + 1, NROW - 1)
        pltpu.async_copy(lp0_hbm.at[nxt], rowA, semA)
        scan_row(rowB, scv1[pl.ds(j * K, K)], j * K, otv1, oti1)
        return carry

    lax.fori_loop(0, RPW, rowloop, 0)
    pltpu.make_async_copy(lp0_hbm.at[0], rowA, semA).wait()

    pltpu.sync_copy(otv0, tv0_hbm.at[pl.ds(base * K, RPW * K)])
    pltpu.sync_copy(oti0, ti0_hbm.at[pl.ds(base * K, RPW * K)])
    pltpu.sync_copy(otv1, tv1_hbm.at[pl.ds(base * K, RPW * K)])
    pltpu.sync_copy(oti1, ti1_hbm.at[pl.ds(base * K, RPW * K)])


def _sc_topk(lp0, lp1, scb0, scb1):
    f32 = jnp.float32
    i32 = jnp.int32
    run = pl.kernel(
        _sc_body,
        out_type=(
            jax.ShapeDtypeStruct((NROW * K,), f32),
            jax.ShapeDtypeStruct((NROW * K,), i32),
            jax.ShapeDtypeStruct((NROW * K,), f32),
            jax.ShapeDtypeStruct((NROW * K,), i32),
        ),
        mesh=plsc.VectorSubcoreMesh(core_axis_name="c", subcore_axis_name="s"),
        compiler_params=pltpu.CompilerParams(needs_layout_passes=False),
        scratch_types=[
            pltpu.VMEM((V,), f32),
            pltpu.VMEM((V,), f32),
            pltpu.VMEM((V // 16,), f32),
            pltpu.VMEM((K,), i32),
            pltpu.VMEM((RPW * K,), f32),
            pltpu.VMEM((RPW * K,), f32),
            pltpu.VMEM((RPW * K,), f32),
            pltpu.VMEM((RPW * K,), i32),
            pltpu.VMEM((RPW * K,), f32),
            pltpu.VMEM((RPW * K,), i32),
            pltpu.SemaphoreType.DMA,
            pltpu.SemaphoreType.DMA,
        ],
    )
    return run(lp0, lp1, scb0, scb1)


def _combine_body(tv0_ref, ti0_ref, tv1_ref, ti1_ref,
                  s0_ref, s1_ref, t0_ref, t1_ref, ib_ref):
    lane4 = jax.lax.broadcasted_iota(jnp.int32, (1, 1, 1, K), 3)

    def rerank(va, ia):
        ve = va[:, :, :, None]
        vf = va[:, :, None, :]
        ie = ia[:, :, :, None]
        if_ = ia[:, :, None, :]
        cond = (vf > ve) | ((vf == ve) & (if_ < ie))
        rank = jnp.sum(cond.astype(jnp.int32), axis=3)
        oh = rank[:, :, :, None] == lane4
        vs = jnp.sum(jnp.where(oh, ve, 0.0), axis=2)
        is_ = jnp.sum(jnp.where(oh, ie, 0), axis=2)
        return vs, is_

    tv0, ti0 = rerank(tv0_ref[...], ti0_ref[...])
    tv1, ti1 = rerank(tv1_ref[...], ti1_ref[...])

    ss = (tv0[:, :, :, None] + tv1[:, :, None, :]).reshape(BSZ, BEAM * K * K)
    fidx = jax.lax.broadcasted_iota(jnp.int32, (BSZ, BEAM * K * K), 1)
    lane16 = jax.lax.broadcasted_iota(jnp.int32, (1, K), 1)

    vacc = jnp.zeros((BSZ, K), jnp.float32)
    iacc = jnp.zeros((BSZ, K), jnp.int32)
    for t in range(K):
        m = jnp.max(ss, axis=1, keepdims=True)
        idx = jnp.min(jnp.where(ss == m, fidx, BIG), axis=1,
                      keepdims=True)
        ss = jnp.where(fidx == idx, NEG, ss)
        oh = lane16 == t
        vacc = vacc + jnp.where(oh, m, 0.0)
        iacc = iacc + jnp.where(oh, idx, 0)

    ib = iacc >> 8
    rem = iacc & 255
    i0 = rem >> 4
    i1 = rem & 15

    beam_i = jax.lax.broadcasted_iota(jnp.int32, (1, 1, BEAM, K), 2)
    col_i = jax.lax.broadcasted_iota(jnp.int32, (1, 1, BEAM, K), 3)
    sel0 = ((ib[:, :, None, None] == beam_i)
            & (i0[:, :, None, None] == col_i))
    sel1 = ((ib[:, :, None, None] == beam_i)
            & (i1[:, :, None, None] == col_i))
    s0_ref[...] = jnp.sum(jnp.where(sel0, tv0[:, None], 0.0), axis=(2, 3))
    s1_ref[...] = jnp.sum(jnp.where(sel1, tv1[:, None], 0.0), axis=(2, 3))
    t0_ref[...] = jnp.sum(jnp.where(sel0, ti0[:, None], 0), axis=(2, 3))
    t1_ref[...] = jnp.sum(jnp.where(sel1, ti1[:, None], 0), axis=(2, 3))
    ib_ref[...] = ib


def _tc_combine(tv0, ti0, tv1, ti1):
    out_shapes = tuple(
        jax.ShapeDtypeStruct((BSZ, K), dt)
        for dt in (jnp.float32, jnp.float32, jnp.int32, jnp.int32, jnp.int32))
    return pl.pallas_call(
        _combine_body,
        out_shape=out_shapes,
    )(tv0, ti0, tv1, ti1)


def kernel(step, lprobs_ch0, lprobs_ch1, scores_ch0, scores_ch1):
    sc0 = jax.lax.dynamic_index_in_dim(scores_ch0, step - 1, axis=2,
                                       keepdims=False)
    sc1 = jax.lax.dynamic_index_in_dim(scores_ch1, step - 1, axis=2,
                                       keepdims=False)
    lp0 = lprobs_ch0.reshape(NROW, V)
    lp1 = lprobs_ch1.reshape(NROW, V)
    scb0 = jnp.broadcast_to(sc0.reshape(NROW, 1), (NROW, K)).reshape(NROW * K)
    scb1 = jnp.broadcast_to(sc1.reshape(NROW, 1), (NROW, K)).reshape(NROW * K)

    tv0, ti0, tv1, ti1 = _sc_topk(lp0, lp1, scb0, scb1)
    tv0 = tv0.reshape(NROW, K)
    ti0 = ti0.reshape(NROW, K)
    tv1 = tv1.reshape(NROW, K)
    ti1 = ti1.reshape(NROW, K)

    s0, s1, t0, t1, ib = _tc_combine(
        tv0.reshape(BSZ, BEAM, K), ti0.reshape(BSZ, BEAM, K),
        tv1.reshape(BSZ, BEAM, K), ti1.reshape(BSZ, BEAM, K))
    return (s0, s1, t0, t1, ib)


# speedup vs baseline: 1.2215x; 1.0000x over previous
import jax
import jax.numpy as jnp
from jax import lax
from jax.experimental import pallas as pl
from jax.experimental.pallas import tpu as pltpu
from jax.experimental.pallas import tpu_sc as plsc

BSZ, BEAM, V = 32, 8, 32768
K = 2 * BEAM
NROW = BSZ * BEAM
NW = 32
RPW = NROW // NW
NGRP = V // 256
NEG = float("-inf")
BIG = 1 << 30


def _sc_body(lp0_hbm, lp1_hbm, scb0_hbm, scb1_hbm,
             tv0_hbm, ti0_hbm, tv1_hbm, ti1_hbm,
             rowA, rowB, gms, tidx_v, scv0, scv1,
             otv0, oti0, otv1, oti1, semA, semB):
    wid = lax.axis_index("s") * 2 + lax.axis_index("c")
    base = wid * RPW
    lane = lax.iota(jnp.int32, K)
    zeros16 = jnp.zeros((K,), jnp.int32)

    pltpu.sync_copy(scb0_hbm.at[pl.ds(base * K, RPW * K)], scv0)
    pltpu.sync_copy(scb1_hbm.at[pl.ds(base * K, RPW * K)], scv1)

    def scan_row(row_v, s, obase, otv, oti):
        @plsc.parallel_loop(0, NGRP, 1, unroll=2)
        def pass_a(g):
            off = g * 256
            ms = [row_v[pl.ds(off + c * K, K)] for c in range(16)]
            while len(ms) > 1:
                ms = [jnp.maximum(ms[i], ms[i + 1])
                      for i in range(0, len(ms), 2)]
            gm = jnp.max(ms[0])
            gms[pl.ds(g * K, K)] = jnp.full((K,), gm, jnp.float32)

        Ts = Ti = None
        for k in range(8):
            gmk = plsc.load_gather(gms, [lane * 17 + 256 * k])
            gik = lane + 16 * k
            sv, si = plsc.sort_key_val(gmk, gik, descending=True)
            if Ts is None:
                Ts, Ti = sv, si
            else:
                bv = lax.rev(sv, (0,))
                bi = lax.rev(si, (0,))
                keep = (Ts > bv) | ((Ts == bv) & (Ti < bi))
                Ts, Ti = plsc.sort_key_val(jnp.where(keep, Ts, bv),
                                           jnp.where(keep, Ti, bi),
                                           descending=True)
        tidx_v[...] = Ti

        def hot(r, carry):
            cv, ci = carry
            gsp = plsc.load_gather(tidx_v, [zeros16 + r])
            g = gsp[0]
            off = g * 256
            leaves = []
            for c in range(16):
                x = row_v[pl.ds(off + c * K, K)] + s
                xi = lane + (off + c * K)
                leaves.append(plsc.sort_key_val(x, xi, descending=True))

            def mrg(a, b):
                av, ai = a
                bv = lax.rev(b[0], (0,))
                bi = lax.rev(b[1], (0,))
                keep = (av > bv) | ((av == bv) & (ai < bi))
                return plsc.sort_key_val(jnp.where(keep, av, bv),
                                         jnp.where(keep, ai, bi),
                                         descending=True)

            while len(leaves) > 1:
                leaves = [mrg(leaves[i], leaves[i + 1])
                          for i in range(0, len(leaves), 2)]
            gv, gi = leaves[0]
            keep = (cv > gv) | ((cv == gv) & (ci < gi))
            mv = jnp.where(keep, cv, gv)
            mi = jnp.where(keep, ci, gi)
            cv, ci = plsc.sort_key_val(mv, mi, descending=False)
            return cv, ci

        cv0 = jnp.full((K,), NEG, jnp.float32)
        ci0 = jnp.zeros((K,), jnp.int32)
        cv, ci = lax.fori_loop(0, 16, hot, (cv0, ci0))
        otv[pl.ds(obase, K)] = lax.rev(cv, (0,))
        oti[pl.ds(obase, K)] = lax.rev(ci, (0,))

    pltpu.async_copy(lp0_hbm.at[base], rowA, semA)

    def rowloop(j, carry):
        row = base + j
        pltpu.make_async_copy(lp0_hbm.at[0], rowA, semA).wait()
        pltpu.async_copy(lp1_hbm.at[row], rowB, semB)
        scan_row(rowA, scv0[pl.ds(j * K, K)], j * K, otv0, oti0)
        pltpu.make_async_copy(lp1_hbm.at[0], rowB, semB).wait()
        nxt = jnp.minimum(row + 1, NROW - 1)
        pltpu.async_copy(lp0_hbm.at[nxt], rowA, semA)
        scan_row(rowB, scv1[pl.ds(j * K, K)], j * K, otv1, oti1)
        return carry

    lax.fori_loop(0, RPW, rowloop, 0)
    pltpu.make_async_copy(lp0_hbm.at[0], rowA, semA).wait()

    pltpu.sync_copy(otv0, tv0_hbm.at[pl.ds(base * K, RPW * K)])
    pltpu.sync_copy(oti0, ti0_hbm.at[pl.ds(base * K, RPW * K)])
    pltpu.sync_copy(otv1, tv1_hbm.at[pl.ds(base * K, RPW * K)])
    pltpu.sync_copy(oti1, ti1_hbm.at[pl.ds(base * K, RPW * K)])


def _sc_topk(lp0, lp1, scb0, scb1):
    f32 = jnp.float32
    i32 = jnp.int32
    run = pl.kernel(
        _sc_body,
        out_type=(
            jax.ShapeDtypeStruct((NROW * K,), f32),
            jax.ShapeDtypeStruct((NROW * K,), i32),
            jax.ShapeDtypeStruct((NROW * K,), f32),
            jax.ShapeDtypeStruct((NROW * K,), i32),
        ),
        mesh=plsc.VectorSubcoreMesh(core_axis_name="c", subcore_axis_name="s"),
        compiler_params=pltpu.CompilerParams(needs_layout_passes=False),
        scratch_types=[
            pltpu.VMEM((V,), f32),
            pltpu.VMEM((V,), f32),
            pltpu.VMEM((V // 16,), f32),
            pltpu.VMEM((K,), i32),
            pltpu.VMEM((RPW * K,), f32),
            pltpu.VMEM((RPW * K,), f32),
            pltpu.VMEM((RPW * K,), f32),
            pltpu.VMEM((RPW * K,), i32),
            pltpu.VMEM((RPW * K,), f32),
            pltpu.VMEM((RPW * K,), i32),
            pltpu.SemaphoreType.DMA,
            pltpu.SemaphoreType.DMA,
        ],
    )
    return run(lp0, lp1, scb0, scb1)


def _combine_body(tv0_ref, ti0_ref, tv1_ref, ti1_ref,
                  s0_ref, s1_ref, t0_ref, t1_ref, ib_ref):
    lane4 = jax.lax.broadcasted_iota(jnp.int32, (1, 1, 1, K), 3)

    def rerank(va, ia):
        ve = va[:, :, :, None]
        vf = va[:, :, None, :]
        ie = ia[:, :, :, None]
        if_ = ia[:, :, None, :]
        cond = (vf > ve) | ((vf == ve) & (if_ < ie))
        rank = jnp.sum(cond.astype(jnp.int32), axis=3)
        oh = rank[:, :, :, None] == lane4
        vs = jnp.sum(jnp.where(oh, ve, 0.0), axis=2)
        is_ = jnp.sum(jnp.where(oh, ie, 0), axis=2)
        return vs, is_

    tv0, ti0 = rerank(tv0_ref[...], ti0_ref[...])
    tv1, ti1 = rerank(tv1_ref[...], ti1_ref[...])

    ss = (tv0[:, :, :, None] + tv1[:, :, None, :]).reshape(BSZ, BEAM * K * K)
    fidx = jax.lax.broadcasted_iota(jnp.int32, (BSZ, BEAM * K * K), 1)
    lane16 = jax.lax.broadcasted_iota(jnp.int32, (1, K), 1)

    vacc = jnp.zeros((BSZ, K), jnp.float32)
    iacc = jnp.zeros((BSZ, K), jnp.int32)
    for t in range(K):
        m = jnp.max(ss, axis=1, keepdims=True)
        idx = jnp.min(jnp.where(ss == m, fidx, BIG), axis=1,
                      keepdims=True)
        ss = jnp.where(fidx == idx, NEG, ss)
        oh = lane16 == t
        vacc = vacc + jnp.where(oh, m, 0.0)
        iacc = iacc + jnp.where(oh, idx, 0)

    ib = iacc >> 8
    rem = iacc & 255
    i0 = rem >> 4
    i1 = rem & 15

    beam_i = jax.lax.broadcasted_iota(jnp.int32, (1, 1, BEAM, K), 2)
    col_i = jax.lax.broadcasted_iota(jnp.int32, (1, 1, BEAM, K), 3)
    sel0 = ((ib[:, :, None, None] == beam_i)
            & (i0[:, :, None, None] == col_i))
    sel1 = ((ib[:, :, None, None] == beam_i)
            & (i1[:, :, None, None] == col_i))
    s0_ref[...] = jnp.sum(jnp.where(sel0, tv0[:, None], 0.0), axis=(2, 3))
    s1_ref[...] = jnp.sum(jnp.where(sel1, tv1[:, None], 0.0), axis=(2, 3))
    t0_ref[...] = jnp.sum(jnp.where(sel0, ti0[:, None], 0), axis=(2, 3))
    t1_ref[...] = jnp.sum(jnp.where(sel1, ti1[:, None], 0), axis=(2, 3))
    ib_ref[...] = ib


def _tc_combine(tv0, ti0, tv1, ti1):
    out_shapes = tuple(
        jax.ShapeDtypeStruct((BSZ, K), dt)
        for dt in (jnp.float32, jnp.float32, jnp.int32, jnp.int32, jnp.int32))
    return pl.pallas_call(
        _combine_body,
        out_shape=out_shapes,
    )(tv0, ti0, tv1, ti1)


def kernel(step, lprobs_ch0, lprobs_ch1, scores_ch0, scores_ch1):
    sc0 = jax.lax.dynamic_index_in_dim(scores_ch0, step - 1, axis=2,
                                       keepdims=False)
    sc1 = jax.lax.dynamic_index_in_dim(scores_ch1, step - 1, axis=2,
                                       keepdims=False)
    lp0 = lprobs_ch0.reshape(NROW, V)
    lp1 = lprobs_ch1.reshape(NROW, V)
    scb0 = jnp.broadcast_to(sc0.reshape(NROW, 1), (NROW, K)).reshape(NROW * K)
    scb1 = jnp.broadcast_to(sc1.reshape(NROW, 1), (NROW, K)).reshape(NROW * K)

    tv0, ti0, tv1, ti1 = _sc_topk(lp0, lp1, scb0, scb1)
    tv0 = tv0.reshape(NROW, K)
    ti0 = ti0.reshape(NROW, K)
    tv1 = tv1.reshape(NROW, K)
    ti1 = ti1.reshape(NROW, K)

    s0, s1, t0, t1, ib = _tc_combine(
        tv0.reshape(BSZ, BEAM, K), ti0.reshape(BSZ, BEAM, K),
        tv1.reshape(BSZ, BEAM, K), ti1.reshape(BSZ, BEAM, K))
    return (s0, s1, t0, t1, ib)
